# baseline (speedup 1.0000x reference)
import functools

import jax
import jax.numpy as jnp
from jax import lax
from jax.experimental import pallas as pl
from jax.experimental.pallas import tpu as pltpu

PROJ_CHUNKS = 4
UNROLL = 2


def _full(shape):
    n = len(shape)
    return pl.BlockSpec(shape, lambda d: (0,) * n)


def _sig(x):
    return 0.5 * jnp.tanh(0.5 * x) + 0.5


def _scan_dir(xg_ref, len_ref, w_hh, d, S, B, H, seq_ref=None):
    f32 = jnp.float32
    bf16 = jnp.bfloat16
    lens = len_ref[...]

    def step(s, carry):
        h, c = carry
        t = lax.select(d == 0, s, S - 1 - s)
        gates = xg_ref[t].astype(f32) + jnp.dot(
            h, w_hh, preferred_element_type=f32)
        pred = lens > t.astype(f32)
        i = _sig(gates[:, 0 * H:1 * H])
        f = _sig(gates[:, 1 * H:2 * H])
        g = jnp.tanh(gates[:, 2 * H:3 * H])
        o = _sig(gates[:, 3 * H:4 * H])
        c_new = f * c + i * g
        h_new = (o * jnp.tanh(c_new)).astype(bf16)
        h = jnp.where(pred, h_new, h)
        c = jnp.where(pred, c_new, c)
        if seq_ref is not None:
            seq_ref[0, t] = h
        return (h, c)

    hz = jnp.zeros((B, H), bf16)
    cz = jnp.zeros((B, H), f32)
    return lax.fori_loop(0, S, step, (hz, cz), unroll=UNROLL)


def _sel(d, a_ref, b_ref):
    return jnp.where(d == 0, a_ref[...], b_ref[...]).astype(jnp.bfloat16)


def _l0_kernel(emb_ref, len_ref, w_ihf_ref, w_ihb_ref, bf_ref, bb_ref,
               w_hhf_ref, w_hhb_ref, seq_ref, xg_ref, *, S, B, H):
    d = pl.program_id(0)
    f32 = jnp.float32
    bf16 = jnp.bfloat16
    w_ih = _sel(d, w_ihf_ref, w_ihb_ref)
    b = jnp.where(d == 0, bf_ref[...], bb_ref[...])
    cs = S // PROJ_CHUNKS
    for k in range(PROJ_CHUNKS):
        sl = pl.ds(k * cs, cs)
        x = emb_ref[sl].reshape(cs * B, -1)
        xg_ref[sl] = (
            jnp.dot(x, w_ih, preferred_element_type=f32) + b
        ).astype(bf16).reshape(cs, B, 4 * H)
    w_hh = _sel(d, w_hhf_ref, w_hhb_ref)
    _scan_dir(xg_ref, len_ref, w_hh, d, S, B, H, seq_ref=seq_ref)


def _l1_kernel(seq_ref, len_ref, w_ihf_ref, w_ihb_ref, bf_ref, bb_ref,
               w_hhf_ref, w_hhb_ref, fcw_ref, out_ref, xg_ref, *, S, B, H):
    d = pl.program_id(0)
    f32 = jnp.float32
    bf16 = jnp.bfloat16
    w_ih = _sel(d, w_ihf_ref, w_ihb_ref)
    b = jnp.where(d == 0, bf_ref[...], bb_ref[...])
    cs = S // PROJ_CHUNKS
    for k in range(PROJ_CHUNKS):
        sl = pl.ds(k * cs, cs)
        sf = seq_ref[0, sl].reshape(cs * B, H)
        sb = seq_ref[1, sl].reshape(cs * B, H)
        xg_ref[sl] = (
            jnp.dot(sf, w_ih[:H], preferred_element_type=f32)
            + jnp.dot(sb, w_ih[H:], preferred_element_type=f32)
            + b
        ).astype(bf16).reshape(cs, B, 4 * H)
    w_hh = _sel(d, w_hhf_ref, w_hhb_ref)
    h, _ = _scan_dir(xg_ref, len_ref, w_hh, d, S, B, H)
    fcw = jnp.where(d == 0, fcw_ref[:H], fcw_ref[H:]).astype(bf16)
    out_ref[0] = jnp.dot(h, fcw, preferred_element_type=f32)


def _forward(text, text_lengths, params):
    bf16 = jnp.bfloat16
    f32 = jnp.float32
    embedded = jnp.take(params["embedding"], text, axis=0).astype(bf16)
    S, B, E = embedded.shape
    H = params["w_hh0f"].shape[0]
    O = params["fc_w"].shape[1]
    lens = text_lengths.astype(f32).reshape(B, 1)

    cparams = pltpu.CompilerParams(
        dimension_semantics=("parallel",),
        vmem_limit_bytes=56 * 1024 * 1024,
    )

    seq = pl.pallas_call(
        functools.partial(_l0_kernel, S=S, B=B, H=H),
        out_shape=jax.ShapeDtypeStruct((2, S, B, H), bf16),
        grid=(2,),
        in_specs=[
            _full((S, B, E)), _full((B, 1)),
            _full((E, 4 * H)), _full((E, 4 * H)),
            _full((1, 4 * H)), _full((1, 4 * H)),
            _full((H, 4 * H)), _full((H, 4 * H)),
        ],
        out_specs=pl.BlockSpec((1, S, B, H), lambda d: (d, 0, 0, 0)),
        scratch_shapes=[pltpu.VMEM((S, B, 4 * H), bf16)],
        compiler_params=cparams,
    )(embedded, lens, params["w_ih0f"], params["w_ih0b"],
      params["b0f"], params["b0b"], params["w_hh0f"], params["w_hh0b"])

    part = pl.pallas_call(
        functools.partial(_l1_kernel, S=S, B=B, H=H),
        out_shape=jax.ShapeDtypeStruct((2, B, O), f32),
        grid=(2,),
        in_specs=[
            _full((2, S, B, H)), _full((B, 1)),
            _full((2 * H, 4 * H)), _full((2 * H, 4 * H)),
            _full((1, 4 * H)), _full((1, 4 * H)),
            _full((H, 4 * H)), _full((H, 4 * H)),
            _full((2 * H, O)),
        ],
        out_specs=pl.BlockSpec((1, B, O), lambda d: (d, 0, 0)),
        scratch_shapes=[pltpu.VMEM((S, B, 4 * H), bf16)],
        compiler_params=cparams,
    )(seq, lens, params["w_ih1f"], params["w_ih1b"],
      params["b1f"], params["b1b"], params["w_hh1f"], params["w_hh1b"],
      params["fc_w"])

    return part[0] + part[1] + params["fc_b"]


def kernel(text, text_lengths, embedding,
           w_ih0f, w_ih0b, w_hh0f, w_hh0b, b0f, b0b,
           w_ih1f, w_ih1b, w_hh1f, w_hh1b, b1f, b1b,
           fc_w, fc_b):
    params = {
        "embedding": embedding,
        "w_ih0f": w_ih0f, "w_ih0b": w_ih0b,
        "w_hh0f": w_hh0f, "w_hh0b": w_hh0b,
        "b0f": b0f, "b0b": b0b,
        "w_ih1f": w_ih1f, "w_ih1b": w_ih1b,
        "w_hh1f": w_hh1f, "w_hh1b": w_hh1b,
        "b1f": b1f, "b1b": b1b,
        "fc_w": fc_w, "fc_b": fc_b,
    }
    return _forward(text, text_lengths, params)
